# Initial kernel scaffold; baseline (speedup 1.0000x reference)
import jax
import jax.numpy as jnp
from jax.experimental import pallas as pl


def kernel(positions, net_to_pin, pin_to_macro, pin_offsets):
    raise NotImplementedError("write your pallas kernel here")


# speedup vs baseline: 22.4778x; 22.4778x over previous
import functools
import math

import jax
import jax.numpy as jnp
from jax import lax
from jax.experimental import pallas as pl
from jax.experimental.pallas import tpu as pltpu
from jax.experimental.pallas import tpu_sc as plsc

GAMMA_F = 10.0
LN2 = 0.6931471805599453

NC, NS, L = 2, 16, 16
NW = NC * NS

B = 4
K = 16
CB = 8
N_NETS = 100000
P_PINS = 400000

NETS_PER_W = 3136
N_PAD = NW * NETS_PER_W
GROUPS = NETS_PER_W // 16
IDX_ROWS_B = NETS_PER_W * K // 128

PINS_PER_W = 12544
P_PAD = NW * PINS_PER_W
CHUNKS_A = PINS_PER_W // 128


def _fast_log(s):
    bits = lax.bitcast_convert_type(s, jnp.int32)
    e = (bits >> 23) - 127
    m = lax.bitcast_convert_type((bits & 0x007FFFFF) | 0x3F800000, jnp.float32)
    z = (m - 1.0) / (m + 1.0)
    z2 = z * z
    p = jnp.float32(1.0 / 7.0)
    p = p * z2 + jnp.float32(1.0 / 5.0)
    p = p * z2 + jnp.float32(1.0 / 3.0)
    p = p * z2 + jnp.float32(1.0)
    return e.astype(jnp.float32) * jnp.float32(LN2) + (2.0 * z) * p


def _worker_id():
    return lax.axis_index("s") * NC + lax.axis_index("c")


def _stage_a_body(pos_t, idx_a, offs, pin_pos, idx_v, gbuf, obuf, wbuf,
                  sem_g, sem_o, sem_w):
    w = _worker_id()
    row0 = w * CHUNKS_A
    pltpu.sync_copy(idx_a.at[w], idx_v)

    iot = lax.iota(jnp.int32, L)
    rowv = [iot + jnp.int32(16 * i) for i in range(8)]
    csplat = [jnp.full((L,), c, jnp.int32) for c in range(8)]

    def issue(j, s):
        base = (row0 + j) * 128
        pltpu.async_copy(pos_t.at[idx_v.at[j]], gbuf.at[s], sem_g.at[s])
        pltpu.async_copy(offs.at[pl.ds(base, 128)], obuf.at[s], sem_o.at[s])

    def wait_in(j, s):
        pltpu.make_async_copy(pos_t.at[idx_v.at[j]], gbuf.at[s],
                              sem_g.at[s]).wait()
        pltpu.make_async_copy(offs.at[pl.ds(0, 128)], obuf.at[s],
                              sem_o.at[s]).wait()

    for s in range(2):
        issue(jnp.int32(s), s)

    def chunk(i, carry, s):
        j = i * 2 + s
        wait_in(j, s)
        @pl.when(j >= 2)
        def _():
            pltpu.make_async_copy(wbuf.at[s],
                                  pin_pos.at[pl.ds(0, 128)], sem_w.at[s]).wait()
        for i8 in range(8):
            offx = plsc.load_gather(obuf.at[s], [rowv[i8], csplat[0]])
            offy = plsc.load_gather(obuf.at[s], [rowv[i8], csplat[1]])
            for c in range(8):
                pv = plsc.load_gather(gbuf.at[s], [rowv[i8], csplat[c]])
                sm = pv + (offx if c % 2 == 0 else offy)
                plsc.store_scatter(wbuf.at[s], [rowv[i8], csplat[c]], sm)
        base = (row0 + j) * 128
        pltpu.async_copy(wbuf.at[s], pin_pos.at[pl.ds(base, 128)], sem_w.at[s])
        @pl.when(j + 2 < CHUNKS_A)
        def _():
            issue(j + 2, s)
        return carry

    def outer(i, carry):
        carry = chunk(i, carry, 0)
        carry = chunk(i, carry, 1)
        return carry

    lax.fori_loop(0, CHUNKS_A // 2, outer, jnp.int32(0))
    for s in range(2):
        pltpu.make_async_copy(wbuf.at[s], pin_pos.at[pl.ds(0, 128)],
                              sem_w.at[s]).wait()


def _stage_b_body(pin_pos, idx_b, out, idx_v, gbuf, outv, sem_g):
    w = _worker_id()
    pltpu.sync_copy(idx_b.at[w], idx_v)

    iot = lax.iota(jnp.int32, L)
    rowk = [iot * 16 + jnp.int32(k) for k in range(K)]
    csplat = [jnp.full((L,), c, jnp.int32) for c in range(CB)]

    def issue(g, s):
        for h in range(2):
            pltpu.async_copy(pin_pos.at[idx_v.at[2 * g + h]],
                             gbuf.at[s, pl.ds(h * 128, 128)], sem_g.at[s])

    def wait_g(s):
        for h in range(2):
            pltpu.make_async_copy(pin_pos.at[pl.ds(0, 128)],
                                  gbuf.at[s, pl.ds(h * 128, 128)],
                                  sem_g.at[s]).wait()

    for s in range(2):
        issue(jnp.int32(s), s)

    def group(i, accs, s):
        g = i * 2 + s
        wait_g(s)
        accs = list(accs)
        for c in range(CB):
            wv = [plsc.load_gather(gbuf.at[s], [rowk[k], csplat[c]])
                  * jnp.float32(GAMMA_F) for k in range(K)]
            wm = wv
            while len(wm) > 1:
                wm = [jnp.maximum(wm[2 * t], wm[2 * t + 1])
                      for t in range(len(wm) // 2)]
            wn = wv
            while len(wn) > 1:
                wn = [jnp.minimum(wn[2 * t], wn[2 * t + 1])
                      for t in range(len(wn) // 2)]
            w_max, w_min = wm[0], wn[0]
            ep = [jnp.exp(v - w_max) for v in wv]
            en = [jnp.exp(w_min - v) for v in wv]
            while len(ep) > 1:
                ep = [ep[2 * t] + ep[2 * t + 1] for t in range(len(ep) // 2)]
            while len(en) > 1:
                en = [en[2 * t] + en[2 * t + 1] for t in range(len(en) // 2)]
            wl = (w_max - w_min + _fast_log(ep[0] * en[0])) \
                * jnp.float32(1.0 / GAMMA_F)
            accs[c // 2] = accs[c // 2] + wl
        @pl.when(g + 2 < GROUPS)
        def _():
            issue(g + 2, s)
        return tuple(accs)

    def outer(i, accs):
        accs = group(i, accs, 0)
        accs = group(i, accs, 1)
        return accs

    zero = jnp.zeros((L,), jnp.float32)
    accs = lax.fori_loop(0, GROUPS // 2, outer, (zero, zero, zero, zero))
    for b in range(B):
        outv[b, :] = accs[b]
    pltpu.sync_copy(outv, out.at[w])


_MESH = plsc.VectorSubcoreMesh(core_axis_name="c", subcore_axis_name="s",
                               num_cores=NC, num_subcores=NS)
_PARAMS = pltpu.CompilerParams(needs_layout_passes=False,
                               use_tc_tiling_on_sc=False)

_stage_a = pl.kernel(
    _stage_a_body,
    out_type=jax.ShapeDtypeStruct((P_PAD, CB), jnp.float32),
    mesh=_MESH,
    compiler_params=_PARAMS,
    scratch_types=[
        pltpu.VMEM((CHUNKS_A, 128), jnp.int32),
        pltpu.VMEM((2, 128, CB), jnp.float32),
        pltpu.VMEM((2, 128, 2), jnp.float32),
        pltpu.VMEM((2, 128, CB), jnp.float32),
        pltpu.SemaphoreType.DMA((2,)),
        pltpu.SemaphoreType.DMA((2,)),
        pltpu.SemaphoreType.DMA((2,)),
    ],
)

_stage_b = pl.kernel(
    _stage_b_body,
    out_type=jax.ShapeDtypeStruct((NW, B, L), jnp.float32),
    mesh=_MESH,
    compiler_params=_PARAMS,
    scratch_types=[
        pltpu.VMEM((IDX_ROWS_B, 128), jnp.int32),
        pltpu.VMEM((2, 256, CB), jnp.float32),
        pltpu.VMEM((B, L), jnp.float32),
        pltpu.SemaphoreType.DMA((2,)),
    ],
)

_PAD_CONST = (N_PAD - N_NETS) * 2.0 * (8.0 * LN2) / GAMMA_F


@jax.jit
def kernel(positions, net_to_pin, pin_to_macro, pin_offsets):
    pos_t = jnp.transpose(positions, (1, 0, 2)).reshape(N_NETS, CB)
    ptm = jnp.pad(pin_to_macro.astype(jnp.int32), (0, P_PAD - P_PINS))
    idx_a = ptm.reshape(NW, CHUNKS_A, 128)
    offs = jnp.pad(pin_offsets, ((0, P_PAD - P_PINS), (0, 0)))
    ntp = jnp.pad(net_to_pin.astype(jnp.int32), ((0, N_PAD - N_NETS), (0, 0)))
    idx_b = ntp.reshape(NW, IDX_ROWS_B, 128)

    pin_pos = _stage_a(pos_t, idx_a, offs)
    partial = _stage_b(pin_pos, idx_b)
    return partial.sum(axis=(0, 2)) - jnp.float32(_PAD_CONST)
